# Initial kernel scaffold; baseline (speedup 1.0000x reference)
import jax
import jax.numpy as jnp
from jax.experimental import pallas as pl


def kernel(x, table):
    raise NotImplementedError("write your pallas kernel here")


# speedup vs baseline: 3.4511x; 3.4511x over previous
import jax
import jax.numpy as jnp
from jax import lax
from jax.experimental import pallas as pl
from jax.experimental.pallas import tpu as pltpu
from jax.experimental.pallas import tpu_sc as plsc
import functools

NC = 2
NS = 16
NW = NC * NS

B = 16384 * 50
D = 128
CH = 256
G = 128
B_PER_W = B // NW
N_PAIR = B_PER_W // (2 * CH)


def _emb_body(idx_hbm, table_hbm, out_hbm,
              idx_a, idx_b, rows_a, rows_b,
              gsem_a, gsem_b, ssem_a, ssem_b):
    wid = lax.axis_index("s") * NC + lax.axis_index("c")
    base = wid * B_PER_W

    @pl.loop(0, N_PAIR)
    def _pair(t):
        off_a = base + t * (2 * CH)
        off_b = off_a + CH

        @pl.when(t > 0)
        def _():
            pltpu.make_async_copy(rows_a, out_hbm.at[pl.ds(0, CH)], ssem_a).wait()

        pltpu.sync_copy(idx_hbm.at[pl.ds(off_a, CH)], idx_a)
        ga0 = pltpu.async_copy(table_hbm.at[idx_a.at[pl.ds(0, G)]],
                               rows_a.at[pl.ds(0, G)], gsem_a)
        ga1 = pltpu.async_copy(table_hbm.at[idx_a.at[pl.ds(G, G)]],
                               rows_a.at[pl.ds(G, G)], gsem_a)

        @pl.when(t > 0)
        def _():
            pltpu.make_async_copy(rows_b, out_hbm.at[pl.ds(0, CH)], ssem_b).wait()

        pltpu.sync_copy(idx_hbm.at[pl.ds(off_b, CH)], idx_b)
        gb0 = pltpu.async_copy(table_hbm.at[idx_b.at[pl.ds(0, G)]],
                               rows_b.at[pl.ds(0, G)], gsem_b)
        gb1 = pltpu.async_copy(table_hbm.at[idx_b.at[pl.ds(G, G)]],
                               rows_b.at[pl.ds(G, G)], gsem_b)

        ga0.wait()
        ga1.wait()
        pltpu.async_copy(rows_a, out_hbm.at[pl.ds(off_a, CH)], ssem_a)
        gb0.wait()
        gb1.wait()
        pltpu.async_copy(rows_b, out_hbm.at[pl.ds(off_b, CH)], ssem_b)

    pltpu.make_async_copy(rows_a, out_hbm.at[pl.ds(0, CH)], ssem_a).wait()
    pltpu.make_async_copy(rows_b, out_hbm.at[pl.ds(0, CH)], ssem_b).wait()


@functools.partial(jax.jit, static_argnames=())
def _emb_lookup(idx_flat, table):
    mesh = plsc.VectorSubcoreMesh(core_axis_name="c", subcore_axis_name="s")
    f = pl.kernel(
        _emb_body,
        out_type=jax.ShapeDtypeStruct((B, D), jnp.float32),
        mesh=mesh,
        scratch_types=[
            pltpu.VMEM((CH,), jnp.int32),
            pltpu.VMEM((CH,), jnp.int32),
            pltpu.VMEM((CH, D), jnp.float32),
            pltpu.VMEM((CH, D), jnp.float32),
            pltpu.SemaphoreType.DMA,
            pltpu.SemaphoreType.DMA,
            pltpu.SemaphoreType.DMA,
            pltpu.SemaphoreType.DMA,
        ],
    )
    return f(idx_flat, table)


def kernel(x, table):
    idx_flat = x.reshape(-1).astype(jnp.int32)
    out = _emb_lookup(idx_flat, table)
    return out.reshape(x.shape[0], x.shape[1], D)


# speedup vs baseline: 5.7011x; 1.6520x over previous
import jax
import jax.numpy as jnp
from jax import lax
from jax.experimental import pallas as pl
from jax.experimental.pallas import tpu as pltpu
from jax.experimental.pallas import tpu_sc as plsc
import functools

NC = 2
NS = 16
NW = NC * NS

R = 16384
S = 50
D = 128
R_PER_W = R // NW
RCH = 8
CH = RCH * S
N_CH = R_PER_W // RCH
G_OFF = (0, 96, 192, 288)
G_LEN = (96, 96, 96, 112)


def _emb_body(idx_hbm, table_hbm, out_hbm, idx_v, rows_v, gsem, ssem):
    wid = lax.axis_index("s") * NC + lax.axis_index("c")
    row0 = wid * R_PER_W

    @pl.loop(0, N_CH)
    def _chunk(t):
        r0 = row0 + t * RCH
        pltpu.sync_copy(idx_hbm.at[pl.ds(r0 * S, CH)], idx_v)
        gs = [pltpu.async_copy(table_hbm.at[idx_v.at[pl.ds(o, n)]],
                               rows_v.at[pl.ds(o, n)], gsem)
              for o, n in zip(G_OFF, G_LEN)]
        for g in gs:
            g.wait()
        ss = [pltpu.async_copy(rows_v.at[pl.ds(r * S, S)],
                               out_hbm.at[r0 + r], ssem)
              for r in range(RCH)]
        for s in ss:
            s.wait()


@functools.partial(jax.jit, static_argnames=())
def _emb_lookup(idx_flat, table):
    mesh = plsc.VectorSubcoreMesh(core_axis_name="c", subcore_axis_name="s")
    f = pl.kernel(
        _emb_body,
        out_type=jax.ShapeDtypeStruct((R, S, D), jnp.float32),
        mesh=mesh,
        scratch_types=[
            pltpu.VMEM((CH,), jnp.int32),
            pltpu.VMEM((CH, D), jnp.float32),
            pltpu.SemaphoreType.DMA,
            pltpu.SemaphoreType.DMA,
        ],
    )
    return f(idx_flat, table)


def kernel(x, table):
    idx_flat = x.reshape(-1).astype(jnp.int32)
    return _emb_lookup(idx_flat, table)


# speedup vs baseline: 6.2100x; 1.0893x over previous
import jax
import jax.numpy as jnp
from jax import lax
from jax.experimental import pallas as pl
from jax.experimental.pallas import tpu as pltpu
from jax.experimental.pallas import tpu_sc as plsc
import functools

NC = 2
NS = 16
NW = NC * NS

R = 16384
S = 50
D = 128
R_PER_W = R // NW
B_PER_W = R_PER_W * S
RCH = 8
CH = RCH * S
N_PAIR = R_PER_W // (2 * RCH)
G_OFF = (0, 96, 192, 288)
G_LEN = (96, 96, 96, 112)


def _emb_body(idx_hbm, table_hbm, out_hbm, idx_v, rows_a, rows_b,
              gsem_a, gsem_b, ssem_a, ssem_b):
    wid = lax.axis_index("s") * NC + lax.axis_index("c")
    row0 = wid * R_PER_W
    pltpu.sync_copy(idx_hbm.at[pl.ds(wid * B_PER_W, B_PER_W)], idx_v)

    def drain_scatters(rows_v, r0, sem):
        for r in range(RCH):
            pltpu.make_async_copy(rows_v.at[pl.ds(r * S, S)],
                                  out_hbm.at[r0 + r], sem).wait()

    def fire_gathers(rows_v, off, sem):
        return [pltpu.async_copy(table_hbm.at[idx_v.at[pl.ds(off + o, n)]],
                                 rows_v.at[pl.ds(o, n)], sem)
                for o, n in zip(G_OFF, G_LEN)]

    def fire_scatters(rows_v, r0, sem):
        for r in range(RCH):
            pltpu.async_copy(rows_v.at[pl.ds(r * S, S)], out_hbm.at[r0 + r],
                             sem)

    @pl.loop(0, N_PAIR)
    def _pair(t):
        ra = row0 + t * (2 * RCH)
        rb = ra + RCH

        @pl.when(t > 0)
        def _():
            drain_scatters(rows_a, ra, ssem_a)
        ga = fire_gathers(rows_a, (t * 2 * RCH) * S, gsem_a)

        @pl.when(t > 0)
        def _():
            drain_scatters(rows_b, rb, ssem_b)
        gb = fire_gathers(rows_b, (t * 2 * RCH + RCH) * S, gsem_b)

        for g in ga:
            g.wait()
        fire_scatters(rows_a, ra, ssem_a)
        for g in gb:
            g.wait()
        fire_scatters(rows_b, rb, ssem_b)

    drain_scatters(rows_a, row0, ssem_a)
    drain_scatters(rows_b, row0 + RCH, ssem_b)


@functools.partial(jax.jit, static_argnames=())
def _emb_lookup(idx_flat, table):
    mesh = plsc.VectorSubcoreMesh(core_axis_name="c", subcore_axis_name="s")
    f = pl.kernel(
        _emb_body,
        out_type=jax.ShapeDtypeStruct((R, S, D), jnp.float32),
        mesh=mesh,
        scratch_types=[
            pltpu.VMEM((B_PER_W,), jnp.int32),
            pltpu.VMEM((CH, D), jnp.float32),
            pltpu.VMEM((CH, D), jnp.float32),
            pltpu.SemaphoreType.DMA,
            pltpu.SemaphoreType.DMA,
            pltpu.SemaphoreType.DMA,
            pltpu.SemaphoreType.DMA,
        ],
    )
    return f(idx_flat, table)


def kernel(x, table):
    idx_flat = x.reshape(-1).astype(jnp.int32)
    return _emb_lookup(idx_flat, table)


# speedup vs baseline: 6.2775x; 1.0109x over previous
import jax
import jax.numpy as jnp
from jax import lax
from jax.experimental import pallas as pl
from jax.experimental.pallas import tpu as pltpu
from jax.experimental.pallas import tpu_sc as plsc
import functools

NC = 2
NS = 16
NW = NC * NS

R = 16384
S = 50
D = 128
R_PER_W = R // NW
B_PER_W = R_PER_W * S
NBUF = 4
RCH = 4
CH = RCH * S
N_GRP = R_PER_W // (NBUF * RCH)
G_OFF = (0, 96)
G_LEN = (96, 104)


def _emb_body(idx_hbm, table_hbm, out_hbm, idx_v,
              rows_0, rows_1, rows_2, rows_3,
              gsem_0, gsem_1, gsem_2, gsem_3,
              ssem_0, ssem_1, ssem_2, ssem_3):
    rows = (rows_0, rows_1, rows_2, rows_3)
    gsem = (gsem_0, gsem_1, gsem_2, gsem_3)
    ssem = (ssem_0, ssem_1, ssem_2, ssem_3)
    wid = lax.axis_index("s") * NC + lax.axis_index("c")
    row0 = wid * R_PER_W
    pltpu.sync_copy(idx_hbm.at[pl.ds(wid * B_PER_W, B_PER_W)], idx_v)

    def drain_scatters(b, r0):
        for r in range(RCH):
            pltpu.make_async_copy(rows[b].at[pl.ds(r * S, S)],
                                  out_hbm.at[r0 + r], ssem[b]).wait()

    def fire_gathers(b, off):
        return [pltpu.async_copy(table_hbm.at[idx_v.at[pl.ds(off + o, n)]],
                                 rows[b].at[pl.ds(o, n)], gsem[b])
                for o, n in zip(G_OFF, G_LEN)]

    def fire_scatters(b, r0):
        for r in range(RCH):
            pltpu.async_copy(rows[b].at[pl.ds(r * S, S)], out_hbm.at[r0 + r],
                             ssem[b])

    @pl.loop(0, N_GRP)
    def _grp(t):
        base = row0 + t * (NBUF * RCH)
        gs = []
        for b in range(NBUF):
            @pl.when(t > 0)
            def _(b=b):
                drain_scatters(b, base + b * RCH)
            gs.append(fire_gathers(b, (t * NBUF + b) * CH))
        for b in range(NBUF):
            for g in gs[b]:
                g.wait()
            fire_scatters(b, base + b * RCH)

    for b in range(NBUF):
        drain_scatters(b, row0 + b * RCH)


@functools.partial(jax.jit, static_argnames=())
def _emb_lookup(idx_flat, table):
    mesh = plsc.VectorSubcoreMesh(core_axis_name="c", subcore_axis_name="s")
    f = pl.kernel(
        _emb_body,
        out_type=jax.ShapeDtypeStruct((R, S, D), jnp.float32),
        mesh=mesh,
        scratch_types=(
            [pltpu.VMEM((B_PER_W,), jnp.int32)]
            + [pltpu.VMEM((CH, D), jnp.float32) for _ in range(NBUF)]
            + [pltpu.SemaphoreType.DMA for _ in range(2 * NBUF)]
        ),
    )
    return f(idx_flat, table)


def kernel(x, table):
    idx_flat = x.reshape(-1).astype(jnp.int32)
    return _emb_lookup(idx_flat, table)
